# Initial kernel scaffold; baseline (speedup 1.0000x reference)
import jax
import jax.numpy as jnp
from jax.experimental import pallas as pl


def kernel(xyz, params):
    raise NotImplementedError("write your pallas kernel here")


# speedup vs baseline: 18.4764x; 18.4764x over previous
import functools

import jax
import jax.numpy as jnp
import numpy as np
from jax.experimental import pallas as pl

_F32 = jnp.float32


def _dotf(a, b):
    return jax.lax.dot_general(a, b, (((1,), (0,)), ((), ())),
                               preferred_element_type=_F32)


def _fps_body(npoint, xs_ref, ys_ref, zs_ref, ox_ref, oy_ref, oz_ref):
    B, N = xs_ref.shape
    xs = xs_ref[...]
    ys = ys_ref[...]
    zs = zs_ref[...]
    lane = jax.lax.broadcasted_iota(jnp.int32, (B, N), 1)
    col = jax.lax.broadcasted_iota(jnp.int32, (B, npoint), 1)

    def step(i, carry):
        dist, far, ax, ay, az = carry
        oh = lane == far
        cx = jnp.sum(jnp.where(oh, xs, 0.0), axis=1, keepdims=True)
        cy = jnp.sum(jnp.where(oh, ys, 0.0), axis=1, keepdims=True)
        cz = jnp.sum(jnp.where(oh, zs, 0.0), axis=1, keepdims=True)
        dx = xs - cx
        dy = ys - cy
        dz = zs - cz
        d = dx * dx + dy * dy + dz * dz
        dist = jnp.minimum(dist, d)
        mv = jnp.max(dist, axis=1, keepdims=True)
        fnew = jnp.min(jnp.where(dist == mv, lane, N), axis=1, keepdims=True)
        sel = col == i
        ax = jnp.where(sel, cx, ax)
        ay = jnp.where(sel, cy, ay)
        az = jnp.where(sel, cz, az)
        return dist, fnew, ax, ay, az

    init = (jnp.full((B, N), 1e10, _F32),
            jnp.zeros((B, 1), jnp.int32),
            jnp.zeros((B, npoint), _F32),
            jnp.zeros((B, npoint), _F32),
            jnp.zeros((B, npoint), _F32))
    _, _, ax, ay, az = jax.lax.fori_loop(0, npoint, step, init)
    ox_ref[...] = ax
    oy_ref[...] = ay
    oz_ref[...] = az


def _fps(xs, ys, zs, npoint):
    B, N = xs.shape
    full2 = lambda shp: pl.BlockSpec(shp, lambda: tuple(0 for _ in shp))
    return pl.pallas_call(
        functools.partial(_fps_body, npoint),
        in_specs=[full2((B, N))] * 3,
        out_specs=[full2((B, npoint))] * 3,
        out_shape=[jax.ShapeDtypeStruct((B, npoint), _F32)] * 3,
    )(xs, ys, zs)


def _pre3_body(f_ref, w0_ref, w1_ref, w2_ref, o0_ref, o1_ref, o2_ref):
    f = f_ref[...]
    o0_ref[...] = _dotf(f, w0_ref[...])
    o1_ref[...] = _dotf(f, w1_ref[...])
    o2_ref[...] = _dotf(f, w2_ref[...])


def _pre3(feat, w0, w1, w2):
    M = feat.shape[0]
    full = lambda shp: pl.BlockSpec(shp, lambda: tuple(0 for _ in shp))
    return pl.pallas_call(
        _pre3_body,
        in_specs=[full(feat.shape), full(w0.shape), full(w1.shape), full(w2.shape)],
        out_specs=[full((M, w0.shape[1])), full((M, w1.shape[1])), full((M, w2.shape[1]))],
        out_shape=[jax.ShapeDtypeStruct((M, w.shape[1]), _F32) for w in (w0, w1, w2)],
    )(feat, w0, w1, w2)


def _group_body(K, r2, cx_ref, cy_ref, cz_ref, xs_ref, ys_ref, zs_ref,
                a_ref, wx_ref, s1_ref, t1_ref, w2_ref, s2_ref, t2_ref,
                w3_ref, s3_ref, t3_ref, o_ref):
    Ts = cx_ref.shape[1]
    N = xs_ref.shape[2]
    C1 = a_ref.shape[2]
    cx = cx_ref[0]
    cy = cy_ref[0]
    cz = cz_ref[0]
    xs = xs_ref[0]
    ys = ys_ref[0]
    zs = zs_ref[0]
    dx = cx - xs
    dy = cy - ys
    dz = cz - zs
    sq = dx * dx + dy * dy + dz * dz
    cond = sq <= r2
    condi = jnp.where(cond, 1, 0).astype(jnp.int32)
    c = condi
    d = 1
    while d < N:
        c = c + jnp.concatenate([jnp.zeros((Ts, d), jnp.int32), c[:, :N - d]],
                                axis=1)
        d *= 2
    rank = c - condi
    count = c[:, N - 1:N]
    kio = jax.lax.broadcasted_iota(jnp.int32, (Ts, K, N), 1)
    P = ((rank[:, None, :] == kio) & cond[:, None, :]).astype(_F32)
    A = a_ref[0]
    H = _dotf(P.reshape(Ts * K, N), A)
    co = cx * wx_ref[0:1, :] + cy * wx_ref[1:2, :] + cz * wx_ref[2:3, :]
    coK = jnp.broadcast_to(co[:, None, :], (Ts, K, C1)).reshape(Ts * K, C1)
    h = jax.nn.relu((H - coK) * s1_ref[...] + t1_ref[...])
    h = jax.nn.relu(_dotf(h, w2_ref[...]) * s2_ref[...] + t2_ref[...])
    h = jax.nn.relu(_dotf(h, w3_ref[...]) * s3_ref[...] + t3_ref[...])
    C3 = h.shape[1]
    h = h.reshape(Ts, K, C3)
    kio2 = jax.lax.broadcasted_iota(jnp.int32, (Ts, K, 1), 1)
    valid = kio2 < count[:, :, None]
    o_ref[0] = jnp.max(jnp.where(valid, h, -jnp.inf), axis=1)


def _group(cents, xs3, ys3, zs3, A, wx, s1, t1, w2, s2, t2, w3, s3, t3,
           K, radius, Ts):
    cx3, cy3, cz3 = cents
    B, S, _ = cx3.shape
    N = xs3.shape[2]
    C1 = A.shape[2]
    C3 = w3.shape[1]
    r2 = np.float32(radius ** 2)
    cspec = pl.BlockSpec((1, Ts, 1), lambda b, j: (b, j, 0))
    pspec = pl.BlockSpec((1, 1, N), lambda b, j: (b, 0, 0))
    aspec = pl.BlockSpec((1, N, C1), lambda b, j: (b, 0, 0))
    wspec = lambda w: pl.BlockSpec(w.shape, lambda b, j: tuple(0 for _ in w.shape))
    return pl.pallas_call(
        functools.partial(_group_body, K, r2),
        grid=(B, S // Ts),
        in_specs=[cspec, cspec, cspec, pspec, pspec, pspec, aspec,
                  wspec(wx), wspec(s1), wspec(t1), wspec(w2), wspec(s2),
                  wspec(t2), wspec(w3), wspec(s3), wspec(t3)],
        out_specs=pl.BlockSpec((1, Ts, C3), lambda b, j: (b, j, 0)),
        out_shape=jax.ShapeDtypeStruct((B, S, C3), _F32),
    )(cx3, cy3, cz3, xs3, ys3, zs3, A, wx, s1, t1, w2, s2, t2, w3, s3, t3)


def _sa3_body(B, S, g_ref, w1_ref, s1_ref, t1_ref, w2_ref, s2_ref, t2_ref,
              w3_ref, s3_ref, t3_ref, wa_ref, sa_ref, ta_ref, wb_ref,
              sb_ref, tb_ref, wc_ref, bc_ref, ox_ref, og_ref, ob_ref):
    h = jax.nn.relu(_dotf(g_ref[...], w1_ref[...]) * s1_ref[...] + t1_ref[...])
    h = jax.nn.relu(_dotf(h, w2_ref[...]) * s2_ref[...] + t2_ref[...])
    h = jax.nn.relu(_dotf(h, w3_ref[...]) * s3_ref[...] + t3_ref[...])
    C = h.shape[1]
    x = jnp.max(h.reshape(B, S, C), axis=1)
    ox_ref[...] = x
    outs = []
    for hd in range(6):
        a = jax.nn.relu(_dotf(x, wa_ref[hd]) * sa_ref[hd:hd + 1, :]
                        + ta_ref[hd:hd + 1, :])
        a = jax.nn.relu(_dotf(a, wb_ref[hd]) * sb_ref[hd:hd + 1, :]
                        + tb_ref[hd:hd + 1, :])
        lg = _dotf(a, wc_ref[hd]) + bc_ref[hd:hd + 1, :]
        m = jnp.max(lg, axis=-1, keepdims=True)
        sh = lg - m
        outs.append(sh - jnp.log(jnp.sum(jnp.exp(sh), axis=-1, keepdims=True)))
    og_ref[...] = outs[0]
    M = outs[1]
    for o in outs[2:]:
        M = jnp.maximum(M, o)
    e = jnp.zeros_like(M)
    for o in outs[1:]:
        e = e + jnp.exp(o - M)
    ob_ref[...] = jnp.log(e) + M


def _sa3_heads(gp, w1, s1, t1, w2, s2, t2, w3, s3, t3,
               wa, sa, ta, wb, sb, tb, wc, bc, B, S):
    full = lambda shp: pl.BlockSpec(shp, lambda: tuple(0 for _ in shp))
    args = (gp, w1, s1, t1, w2, s2, t2, w3, s3, t3, wa, sa, ta, wb, sb, tb, wc, bc)
    C = w3.shape[1]
    NC = wc.shape[2]
    return pl.pallas_call(
        functools.partial(_sa3_body, B, S),
        in_specs=[full(a.shape) for a in args],
        out_specs=[full((B, C)), full((B, NC)), full((B, NC))],
        out_shape=[jax.ShapeDtypeStruct((B, C), _F32),
                   jax.ShapeDtypeStruct((B, NC), _F32),
                   jax.ShapeDtypeStruct((B, NC), _F32)],
    )(*args)


def _fold(lp):
    s = lp['gamma'] / jnp.sqrt(lp['var'] + 1e-5)
    t = (lp['b'] - lp['mean']) * s + lp['beta']
    return lp['W'].T, s[None, :], t[None, :]


def _sa_branch(cents, xs3, ys3, zs3, A, layers, K, radius, Ts, cin):
    w1t, s1, t1 = _fold(layers[0])
    wx = w1t[cin - 3:, :]
    w2t, s2, t2 = _fold(layers[1])
    w3t, s3, t3 = _fold(layers[2])
    return _group(cents, xs3, ys3, zs3, A, wx, s1, t1, w2t, s2, t2,
                  w3t, s3, t3, K, radius, Ts)


def kernel(xyz, params):
    B, _, N = xyz.shape
    S1, S2 = 512, 128

    pts = xyz[:, :3, :]
    xs1, ys1, zs1 = pts[:, 0], pts[:, 1], pts[:, 2]
    feat6 = jnp.concatenate([jnp.transpose(xyz[:, 3:6, :], (0, 2, 1)),
                             jnp.transpose(pts, (0, 2, 1))], axis=-1)
    feat6 = feat6.reshape(B * N, 6)

    sa1 = params['sa1']
    w1ts = [_fold(br[0])[0] for br in sa1]
    A0, A1, A2 = _pre3(feat6, *w1ts)
    nx, ny, nz = _fps(xs1, ys1, zs1, S1)
    cents1 = (nx[:, :, None], ny[:, :, None], nz[:, :, None])
    p3 = lambda a: a.reshape(B, 1, N)
    specs1 = [(16, 0.1, 32), (32, 0.2, 16), (128, 0.4, 8)]
    outs1 = []
    for (K, r, Ts), br, A in zip(specs1, sa1, (A0, A1, A2)):
        Ar = A.reshape(B, N, A.shape[1])
        outs1.append(_sa_branch(cents1, p3(xs1), p3(ys1), p3(zs1), Ar,
                                br, K, r, Ts, 6))
    l1_points = jnp.concatenate(outs1, axis=-1)

    sa2 = params['sa2']
    l1_xyz = jnp.stack([nx, ny, nz], axis=-1)
    feat323 = jnp.concatenate([l1_points, l1_xyz], axis=-1).reshape(B * S1, 323)
    w2ts = [_fold(br[0])[0] for br in sa2]
    B0, B1, B2 = _pre3(feat323, *w2ts)
    mx, my, mz = _fps(nx, ny, nz, S2)
    cents2 = (mx[:, :, None], my[:, :, None], mz[:, :, None])
    q3 = lambda a: a.reshape(B, 1, S1)
    specs2 = [(32, 0.2, 32), (64, 0.4, 16), (128, 0.8, 8)]
    outs2 = []
    for (K, r, Ts), br, A in zip(specs2, sa2, (B0, B1, B2)):
        Ar = A.reshape(B, S1, A.shape[1])
        outs2.append(_sa_branch(cents2, q3(nx), q3(ny), q3(nz), Ar,
                                br, K, r, Ts, 323))
    l2_points = jnp.concatenate(outs2, axis=-1)

    l2xy = jnp.stack([mx, my], axis=-1)
    gp = jnp.concatenate([l2xy, l2_points], axis=-1).reshape(B * S2, 642)
    sa3 = params['sa3']
    w1, s1, t1 = _fold(sa3[0])
    w2, s2, t2 = _fold(sa3[1])
    w3, s3, t3 = _fold(sa3[2])
    heads = ['g2', 'b1', 'b2', 'b3', 'b4', 'b5']
    cls = params['cls']
    wa = jnp.stack([cls[h][0]['W'].T for h in heads], 0)
    sa_ = jnp.stack([_fold(cls[h][0])[1][0] for h in heads], 0)
    ta_ = jnp.stack([_fold(cls[h][0])[2][0] for h in heads], 0)
    wb = jnp.stack([cls[h][1]['W'].T for h in heads], 0)
    sb_ = jnp.stack([_fold(cls[h][1])[1][0] for h in heads], 0)
    tb_ = jnp.stack([_fold(cls[h][1])[2][0] for h in heads], 0)
    wc = jnp.stack([cls[h][2]['W'].T for h in heads], 0)
    bc = jnp.stack([cls[h][2]['b'] for h in heads], 0)
    x, x_g2, x_b = _sa3_heads(gp, w1, s1, t1, w2, s2, t2, w3, s3, t3,
                              wa, sa_, ta_, wb, sb_, tb_, wc, bc, B, S2)

    l3_points = x[:, :, None]
    g = cls['g2']
    g_w = (g[0]['W'], g[1]['W'], g[2]['W'])
    bkeys = heads[1:]
    b_w = tuple(jnp.transpose(jnp.stack([cls[k][i]['W'] for k in bkeys], 0),
                              (1, 0, 2)) for i in (0, 1, 2))
    return (x_g2, l3_points, x_b, g_w, b_w)


# speedup vs baseline: 24.0967x; 1.3042x over previous
import functools

import jax
import jax.numpy as jnp
import numpy as np
from jax.experimental import pallas as pl

_F32 = jnp.float32


def _dotf(a, b):
    return jax.lax.dot_general(a, b, (((1,), (0,)), ((), ())),
                               preferred_element_type=_F32)


def _fps_body(npoint, xs_ref, ys_ref, zs_ref, ox_ref, oy_ref, oz_ref):
    B, N = xs_ref.shape
    xs = xs_ref[...]
    ys = ys_ref[...]
    zs = zs_ref[...]
    lane = jax.lax.broadcasted_iota(jnp.int32, (B, N), 1)
    col = jax.lax.broadcasted_iota(jnp.int32, (B, npoint), 1)

    def step(i, carry):
        dist, far, ax, ay, az = carry
        oh = lane == far
        cx = jnp.sum(jnp.where(oh, xs, 0.0), axis=1, keepdims=True)
        cy = jnp.sum(jnp.where(oh, ys, 0.0), axis=1, keepdims=True)
        cz = jnp.sum(jnp.where(oh, zs, 0.0), axis=1, keepdims=True)
        dx = xs - cx
        dy = ys - cy
        dz = zs - cz
        d = dx * dx + dy * dy + dz * dz
        dist = jnp.minimum(dist, d)
        mv = jnp.max(dist, axis=1, keepdims=True)
        fnew = jnp.min(jnp.where(dist == mv, lane, N), axis=1, keepdims=True)
        sel = col == i
        ax = jnp.where(sel, cx, ax)
        ay = jnp.where(sel, cy, ay)
        az = jnp.where(sel, cz, az)
        return dist, fnew, ax, ay, az

    init = (jnp.full((B, N), 1e10, _F32),
            jnp.zeros((B, 1), jnp.int32),
            jnp.zeros((B, npoint), _F32),
            jnp.zeros((B, npoint), _F32),
            jnp.zeros((B, npoint), _F32))
    _, _, ax, ay, az = jax.lax.fori_loop(0, npoint, step, init)
    ox_ref[...] = ax
    oy_ref[...] = ay
    oz_ref[...] = az


def _fps(xs, ys, zs, npoint):
    B, N = xs.shape
    full2 = lambda shp: pl.BlockSpec(shp, lambda: tuple(0 for _ in shp))
    return pl.pallas_call(
        functools.partial(_fps_body, npoint),
        in_specs=[full2((B, N))] * 3,
        out_specs=[full2((B, npoint))] * 3,
        out_shape=[jax.ShapeDtypeStruct((B, npoint), _F32)] * 3,
    )(xs, ys, zs)


def _pre3_body(f_ref, w0_ref, w1_ref, w2_ref, o0_ref, o1_ref, o2_ref):
    f = f_ref[...]
    o0_ref[...] = _dotf(f, w0_ref[...])
    o1_ref[...] = _dotf(f, w1_ref[...])
    o2_ref[...] = _dotf(f, w2_ref[...])


def _pre3(feat, w0, w1, w2):
    M = feat.shape[0]
    full = lambda shp: pl.BlockSpec(shp, lambda: tuple(0 for _ in shp))
    return pl.pallas_call(
        _pre3_body,
        in_specs=[full(feat.shape), full(w0.shape), full(w1.shape), full(w2.shape)],
        out_specs=[full((M, w0.shape[1])), full((M, w1.shape[1])), full((M, w2.shape[1]))],
        out_shape=[jax.ShapeDtypeStruct((M, w.shape[1]), _F32) for w in (w0, w1, w2)],
    )(feat, w0, w1, w2)


def _group3_body(Ks, r2s, *refs):
    (cx_ref, cy_ref, cz_ref, xs_ref, ys_ref, zs_ref) = refs[:6]
    a_refs = refs[6:9]
    w_refs = [refs[9 + 9 * i: 9 + 9 * (i + 1)] for i in range(3)]
    o_refs = refs[36:39]
    Ts = cx_ref.shape[1]
    N = xs_ref.shape[2]
    cx = cx_ref[0]
    cy = cy_ref[0]
    cz = cz_ref[0]
    dx = cx - xs_ref[0]
    dy = cy - ys_ref[0]
    dz = cz - zs_ref[0]
    sq = dx * dx + dy * dy + dz * dz
    for br in range(3):
        K = Ks[br]
        r2 = r2s[br]
        a_ref = a_refs[br]
        (wx_ref, s1_ref, t1_ref, w2_ref, s2_ref, t2_ref,
         w3_ref, s3_ref, t3_ref) = w_refs[br]
        C1 = a_ref.shape[2]
        cond = sq <= r2
        condi = jnp.where(cond, 1, 0).astype(jnp.int32)
        c = condi
        d = 1
        while d < N:
            c = c + jnp.concatenate(
                [jnp.zeros((Ts, d), jnp.int32), c[:, :N - d]], axis=1)
            d *= 2
        count = c[:, N - 1:N]
        rankm = jnp.where(cond, c - condi, K)
        kio = jax.lax.broadcasted_iota(jnp.int32, (Ts, K, N), 1)
        P = (rankm[:, None, :] == kio).astype(_F32)
        H = _dotf(P.reshape(Ts * K, N), a_ref[0])
        co = (cx * wx_ref[0:1, :] + cy * wx_ref[1:2, :]
              + cz * wx_ref[2:3, :])
        coK = jnp.broadcast_to(co[:, None, :], (Ts, K, C1)).reshape(Ts * K, C1)
        h = jax.nn.relu((H - coK) * s1_ref[...] + t1_ref[...])
        h = jax.nn.relu(_dotf(h, w2_ref[...]) * s2_ref[...] + t2_ref[...])
        h = jax.nn.relu(_dotf(h, w3_ref[...]) * s3_ref[...] + t3_ref[...])
        C3 = h.shape[1]
        h = h.reshape(Ts, K, C3)
        kio2 = jax.lax.broadcasted_iota(jnp.int32, (Ts, K, 1), 1)
        valid = kio2 < count[:, :, None]
        o_refs[br][0] = jnp.max(jnp.where(valid, h, -jnp.inf), axis=1)


def _group3(cents, xs3, ys3, zs3, As, branches, Ks, radii, Ts):
    cx3, cy3, cz3 = cents
    B, S, _ = cx3.shape
    N = xs3.shape[2]
    r2s = tuple(np.float32(r ** 2) for r in radii)
    cspec = pl.BlockSpec((1, Ts, 1), lambda b, j: (b, j, 0))
    pspec = pl.BlockSpec((1, 1, N), lambda b, j: (b, 0, 0))
    wspec = lambda w: pl.BlockSpec(w.shape, lambda b, j: tuple(0 for _ in w.shape))
    in_specs = [cspec, cspec, cspec, pspec, pspec, pspec]
    args = [cx3, cy3, cz3, xs3, ys3, zs3]
    for A in As:
        in_specs.append(pl.BlockSpec((1, N, A.shape[2]), lambda b, j: (b, 0, 0)))
        args.append(A)
    C3s = []
    for br, lws in enumerate(branches):
        for w in lws:
            in_specs.append(wspec(w))
            args.append(w)
        C3s.append(lws[6].shape[1])
    out_specs = [pl.BlockSpec((1, Ts, C3), lambda b, j: (b, j, 0)) for C3 in C3s]
    out_shape = [jax.ShapeDtypeStruct((B, S, C3), _F32) for C3 in C3s]
    return pl.pallas_call(
        functools.partial(_group3_body, Ks, r2s),
        grid=(B, S // Ts),
        in_specs=in_specs,
        out_specs=out_specs,
        out_shape=out_shape,
    )(*args)


def _sa3_body(B, S, g_ref, w1_ref, s1_ref, t1_ref, w2_ref, s2_ref, t2_ref,
              w3_ref, s3_ref, t3_ref, wa_ref, sa_ref, ta_ref, wb_ref,
              sb_ref, tb_ref, wc_ref, bc_ref, ox_ref, og_ref, ob_ref):
    h = jax.nn.relu(_dotf(g_ref[...], w1_ref[...]) * s1_ref[...] + t1_ref[...])
    h = jax.nn.relu(_dotf(h, w2_ref[...]) * s2_ref[...] + t2_ref[...])
    h = jax.nn.relu(_dotf(h, w3_ref[...]) * s3_ref[...] + t3_ref[...])
    C = h.shape[1]
    x = jnp.max(h.reshape(B, S, C), axis=1)
    ox_ref[...] = x
    outs = []
    for hd in range(6):
        a = jax.nn.relu(_dotf(x, wa_ref[hd]) * sa_ref[hd:hd + 1, :]
                        + ta_ref[hd:hd + 1, :])
        a = jax.nn.relu(_dotf(a, wb_ref[hd]) * sb_ref[hd:hd + 1, :]
                        + tb_ref[hd:hd + 1, :])
        lg = _dotf(a, wc_ref[hd]) + bc_ref[hd:hd + 1, :]
        m = jnp.max(lg, axis=-1, keepdims=True)
        sh = lg - m
        outs.append(sh - jnp.log(jnp.sum(jnp.exp(sh), axis=-1, keepdims=True)))
    og_ref[...] = outs[0]
    M = outs[1]
    for o in outs[2:]:
        M = jnp.maximum(M, o)
    e = jnp.zeros_like(M)
    for o in outs[1:]:
        e = e + jnp.exp(o - M)
    ob_ref[...] = jnp.log(e) + M


def _sa3_heads(gp, w1, s1, t1, w2, s2, t2, w3, s3, t3,
               wa, sa, ta, wb, sb, tb, wc, bc, B, S):
    full = lambda shp: pl.BlockSpec(shp, lambda: tuple(0 for _ in shp))
    args = (gp, w1, s1, t1, w2, s2, t2, w3, s3, t3, wa, sa, ta, wb, sb, tb, wc, bc)
    C = w3.shape[1]
    NC = wc.shape[2]
    return pl.pallas_call(
        functools.partial(_sa3_body, B, S),
        in_specs=[full(a.shape) for a in args],
        out_specs=[full((B, C)), full((B, NC)), full((B, NC))],
        out_shape=[jax.ShapeDtypeStruct((B, C), _F32),
                   jax.ShapeDtypeStruct((B, NC), _F32),
                   jax.ShapeDtypeStruct((B, NC), _F32)],
    )(*args)


def _fold(lp):
    s = lp['gamma'] / jnp.sqrt(lp['var'] + 1e-5)
    t = (lp['b'] - lp['mean']) * s + lp['beta']
    return lp['W'].T, s[None, :], t[None, :]


def _branch_weights(layers, cin):
    w1t, s1, t1 = _fold(layers[0])
    wx = w1t[cin - 3:, :]
    w2t, s2, t2 = _fold(layers[1])
    w3t, s3, t3 = _fold(layers[2])
    return (wx, s1, t1, w2t, s2, t2, w3t, s3, t3)


def kernel(xyz, params):
    B, _, N = xyz.shape
    S1, S2 = 512, 128

    pts = xyz[:, :3, :]
    xs1, ys1, zs1 = pts[:, 0], pts[:, 1], pts[:, 2]
    feat6 = jnp.concatenate([jnp.transpose(xyz[:, 3:6, :], (0, 2, 1)),
                             jnp.transpose(pts, (0, 2, 1))], axis=-1)
    feat6 = feat6.reshape(B * N, 6)

    sa1 = params['sa1']
    w1ts = [_fold(br[0])[0] for br in sa1]
    A0, A1, A2 = _pre3(feat6, *w1ts)
    As1 = [a.reshape(B, N, a.shape[1]) for a in (A0, A1, A2)]
    nx, ny, nz = _fps(xs1, ys1, zs1, S1)
    cents1 = (nx[:, :, None], ny[:, :, None], nz[:, :, None])
    p3 = lambda a: a.reshape(B, 1, N)
    br1 = [_branch_weights(br, 6) for br in sa1]
    outs1 = _group3(cents1, p3(xs1), p3(ys1), p3(zs1), As1, br1,
                    (16, 32, 128), (0.1, 0.2, 0.4), 16)
    l1_points = jnp.concatenate(outs1, axis=-1)

    sa2 = params['sa2']
    l1_xyz = jnp.stack([nx, ny, nz], axis=-1)
    feat323 = jnp.concatenate([l1_points, l1_xyz], axis=-1).reshape(B * S1, 323)
    w2ts = [_fold(br[0])[0] for br in sa2]
    B0, B1, B2 = _pre3(feat323, *w2ts)
    As2 = [a.reshape(B, S1, a.shape[1]) for a in (B0, B1, B2)]
    mx, my, mz = _fps(nx, ny, nz, S2)
    cents2 = (mx[:, :, None], my[:, :, None], mz[:, :, None])
    q3 = lambda a: a.reshape(B, 1, S1)
    br2 = [_branch_weights(br, 323) for br in sa2]
    outs2 = _group3(cents2, q3(nx), q3(ny), q3(nz), As2, br2,
                    (32, 64, 128), (0.2, 0.4, 0.8), 16)
    l2_points = jnp.concatenate(outs2, axis=-1)

    l2xy = jnp.stack([mx, my], axis=-1)
    gp = jnp.concatenate([l2xy, l2_points], axis=-1).reshape(B * S2, 642)
    sa3 = params['sa3']
    w1, s1, t1 = _fold(sa3[0])
    w2, s2, t2 = _fold(sa3[1])
    w3, s3, t3 = _fold(sa3[2])
    heads = ['g2', 'b1', 'b2', 'b3', 'b4', 'b5']
    cls = params['cls']
    wa = jnp.stack([cls[h][0]['W'].T for h in heads], 0)
    sa_ = jnp.stack([_fold(cls[h][0])[1][0] for h in heads], 0)
    ta_ = jnp.stack([_fold(cls[h][0])[2][0] for h in heads], 0)
    wb = jnp.stack([cls[h][1]['W'].T for h in heads], 0)
    sb_ = jnp.stack([_fold(cls[h][1])[1][0] for h in heads], 0)
    tb_ = jnp.stack([_fold(cls[h][1])[2][0] for h in heads], 0)
    wc = jnp.stack([cls[h][2]['W'].T for h in heads], 0)
    bc = jnp.stack([cls[h][2]['b'] for h in heads], 0)
    x, x_g2, x_b = _sa3_heads(gp, w1, s1, t1, w2, s2, t2, w3, s3, t3,
                              wa, sa_, ta_, wb, sb_, tb_, wc, bc, B, S2)

    l3_points = x[:, :, None]
    g = cls['g2']
    g_w = (g[0]['W'], g[1]['W'], g[2]['W'])
    bkeys = heads[1:]
    b_w = tuple(jnp.transpose(jnp.stack([cls[k][i]['W'] for k in bkeys], 0),
                              (1, 0, 2)) for i in (0, 1, 2))
    return (x_g2, l3_points, x_b, g_w, b_w)


# speedup vs baseline: 26.6037x; 1.1040x over previous
import functools

import jax
import jax.numpy as jnp
import numpy as np
from jax.experimental import pallas as pl

_F32 = jnp.float32


def _dotf(a, b):
    return jax.lax.dot_general(a, b, (((1,), (0,)), ((), ())),
                               preferred_element_type=_F32)


def _fps_body(npoint, xs_ref, ys_ref, zs_ref, ox_ref, oy_ref, oz_ref):
    B, N = xs_ref.shape
    xs = xs_ref[...]
    ys = ys_ref[...]
    zs = zs_ref[...]
    lane = jax.lax.broadcasted_iota(jnp.int32, (B, N), 1)
    col = jax.lax.broadcasted_iota(jnp.int32, (B, npoint), 1)

    def step(i, carry):
        dist, far, ax, ay, az = carry
        oh = lane == far
        cx = jnp.sum(jnp.where(oh, xs, 0.0), axis=1, keepdims=True)
        cy = jnp.sum(jnp.where(oh, ys, 0.0), axis=1, keepdims=True)
        cz = jnp.sum(jnp.where(oh, zs, 0.0), axis=1, keepdims=True)
        dx = xs - cx
        dy = ys - cy
        dz = zs - cz
        d = dx * dx + dy * dy + dz * dz
        dist = jnp.minimum(dist, d)
        mv = jnp.max(dist, axis=1, keepdims=True)
        fnew = jnp.min(jnp.where(dist == mv, lane, N), axis=1, keepdims=True)
        sel = col == i
        ax = jnp.where(sel, cx, ax)
        ay = jnp.where(sel, cy, ay)
        az = jnp.where(sel, cz, az)
        return dist, fnew, ax, ay, az

    init = (jnp.full((B, N), 1e10, _F32),
            jnp.zeros((B, 1), jnp.int32),
            jnp.zeros((B, npoint), _F32),
            jnp.zeros((B, npoint), _F32),
            jnp.zeros((B, npoint), _F32))
    _, _, ax, ay, az = jax.lax.fori_loop(0, npoint, step, init)
    ox_ref[...] = ax
    oy_ref[...] = ay
    oz_ref[...] = az


def _fps(xs, ys, zs, npoint):
    B, N = xs.shape
    full2 = lambda shp: pl.BlockSpec(shp, lambda: tuple(0 for _ in shp))
    return pl.pallas_call(
        functools.partial(_fps_body, npoint),
        in_specs=[full2((B, N))] * 3,
        out_specs=[full2((B, npoint))] * 3,
        out_shape=[jax.ShapeDtypeStruct((B, npoint), _F32)] * 3,
    )(xs, ys, zs)


def _pre3_body(f_ref, w0_ref, w1_ref, w2_ref, o0_ref, o1_ref, o2_ref):
    f = f_ref[...]
    o0_ref[...] = _dotf(f, w0_ref[...])
    o1_ref[...] = _dotf(f, w1_ref[...])
    o2_ref[...] = _dotf(f, w2_ref[...])


def _pre3(feat, w0, w1, w2):
    M = feat.shape[0]
    full = lambda shp: pl.BlockSpec(shp, lambda: tuple(0 for _ in shp))
    return pl.pallas_call(
        _pre3_body,
        in_specs=[full(feat.shape), full(w0.shape), full(w1.shape), full(w2.shape)],
        out_specs=[full((M, w0.shape[1])), full((M, w1.shape[1])), full((M, w2.shape[1]))],
        out_shape=[jax.ShapeDtypeStruct((M, w.shape[1]), _F32) for w in (w0, w1, w2)],
    )(feat, w0, w1, w2)


def _group3_body(Ks, r2s, *refs):
    (cx_ref, cy_ref, cz_ref, xs_ref, ys_ref, zs_ref) = refs[:6]
    a_refs = refs[6:9]
    w_refs = [refs[9 + 9 * i: 9 + 9 * (i + 1)] for i in range(3)]
    o_refs = refs[36:39]
    Ts = cx_ref.shape[1]
    N = xs_ref.shape[2]
    cx = cx_ref[0]
    cy = cy_ref[0]
    cz = cz_ref[0]
    dx = cx - xs_ref[0]
    dy = cy - ys_ref[0]
    dz = cz - zs_ref[0]
    sq = dx * dx + dy * dy + dz * dz
    for br in range(3):
        K = Ks[br]
        r2 = r2s[br]
        a_ref = a_refs[br]
        (wx_ref, s1_ref, t1_ref, w2_ref, s2_ref, t2_ref,
         w3_ref, s3_ref, t3_ref) = w_refs[br]
        C1 = a_ref.shape[2]
        cond = sq <= r2
        condi = jnp.where(cond, 1, 0).astype(jnp.int32)
        c = condi
        d = 1
        while d < N:
            c = c + jnp.concatenate(
                [jnp.zeros((Ts, d), jnp.int32), c[:, :N - d]], axis=1)
            d *= 2
        count = c[:, N - 1:N]
        rankm = jnp.where(cond, c - condi, K)
        kio = jax.lax.broadcasted_iota(jnp.int32, (Ts, K, N), 1)
        P = (rankm[:, None, :] == kio).astype(_F32)
        H = _dotf(P.reshape(Ts * K, N), a_ref[0])
        co = (cx * wx_ref[0:1, :] + cy * wx_ref[1:2, :]
              + cz * wx_ref[2:3, :])
        coK = jnp.broadcast_to(co[:, None, :], (Ts, K, C1)).reshape(Ts * K, C1)
        h = jax.nn.relu((H - coK) * s1_ref[...] + t1_ref[...])
        h = jax.nn.relu(_dotf(h, w2_ref[...]) * s2_ref[...] + t2_ref[...])
        h = jax.nn.relu(_dotf(h, w3_ref[...]) * s3_ref[...] + t3_ref[...])
        C3 = h.shape[1]
        h = h.reshape(Ts, K, C3)
        kio2 = jax.lax.broadcasted_iota(jnp.int32, (Ts, K, 1), 1)
        valid = kio2 < count[:, :, None]
        o_refs[br][0] = jnp.max(jnp.where(valid, h, -jnp.inf), axis=1)


def _group3(cents, xs3, ys3, zs3, As, branches, Ks, radii, Ts):
    cx3, cy3, cz3 = cents
    B, S, _ = cx3.shape
    N = xs3.shape[2]
    r2s = tuple(np.float32(r ** 2) for r in radii)
    cspec = pl.BlockSpec((1, Ts, 1), lambda b, j: (b, j, 0))
    pspec = pl.BlockSpec((1, 1, N), lambda b, j: (b, 0, 0))
    wspec = lambda w: pl.BlockSpec(w.shape, lambda b, j: tuple(0 for _ in w.shape))
    in_specs = [cspec, cspec, cspec, pspec, pspec, pspec]
    args = [cx3, cy3, cz3, xs3, ys3, zs3]
    for A in As:
        in_specs.append(pl.BlockSpec((1, N, A.shape[2]), lambda b, j: (b, 0, 0)))
        args.append(A)
    C3s = []
    for br, lws in enumerate(branches):
        for w in lws:
            in_specs.append(wspec(w))
            args.append(w)
        C3s.append(lws[6].shape[1])
    out_specs = [pl.BlockSpec((1, Ts, C3), lambda b, j: (b, j, 0)) for C3 in C3s]
    out_shape = [jax.ShapeDtypeStruct((B, S, C3), _F32) for C3 in C3s]
    return pl.pallas_call(
        functools.partial(_group3_body, Ks, r2s),
        grid=(B, S // Ts),
        in_specs=in_specs,
        out_specs=out_specs,
        out_shape=out_shape,
    )(*args)


def _sa3_body(B, S, g_ref, w1_ref, s1_ref, t1_ref, w2_ref, s2_ref, t2_ref,
              w3_ref, s3_ref, t3_ref, wa_ref, sa_ref, ta_ref, wb_ref,
              sb_ref, tb_ref, wc_ref, bc_ref, ox_ref, og_ref, ob_ref):
    h = jax.nn.relu(_dotf(g_ref[...], w1_ref[...]) * s1_ref[...] + t1_ref[...])
    h = jax.nn.relu(_dotf(h, w2_ref[...]) * s2_ref[...] + t2_ref[...])
    h = jax.nn.relu(_dotf(h, w3_ref[...]) * s3_ref[...] + t3_ref[...])
    C = h.shape[1]
    x = jnp.max(h.reshape(B, S, C), axis=1)
    ox_ref[...] = x
    outs = []
    for hd in range(6):
        a = jax.nn.relu(_dotf(x, wa_ref[hd]) * sa_ref[hd:hd + 1, :]
                        + ta_ref[hd:hd + 1, :])
        a = jax.nn.relu(_dotf(a, wb_ref[hd]) * sb_ref[hd:hd + 1, :]
                        + tb_ref[hd:hd + 1, :])
        lg = _dotf(a, wc_ref[hd]) + bc_ref[hd:hd + 1, :]
        m = jnp.max(lg, axis=-1, keepdims=True)
        sh = lg - m
        outs.append(sh - jnp.log(jnp.sum(jnp.exp(sh), axis=-1, keepdims=True)))
    og_ref[...] = outs[0]
    M = outs[1]
    for o in outs[2:]:
        M = jnp.maximum(M, o)
    e = jnp.zeros_like(M)
    for o in outs[1:]:
        e = e + jnp.exp(o - M)
    ob_ref[...] = jnp.log(e) + M


def _sa3_heads(gp, w1, s1, t1, w2, s2, t2, w3, s3, t3,
               wa, sa, ta, wb, sb, tb, wc, bc, B, S):
    full = lambda shp: pl.BlockSpec(shp, lambda: tuple(0 for _ in shp))
    args = (gp, w1, s1, t1, w2, s2, t2, w3, s3, t3, wa, sa, ta, wb, sb, tb, wc, bc)
    C = w3.shape[1]
    NC = wc.shape[2]
    return pl.pallas_call(
        functools.partial(_sa3_body, B, S),
        in_specs=[full(a.shape) for a in args],
        out_specs=[full((B, C)), full((B, NC)), full((B, NC))],
        out_shape=[jax.ShapeDtypeStruct((B, C), _F32),
                   jax.ShapeDtypeStruct((B, NC), _F32),
                   jax.ShapeDtypeStruct((B, NC), _F32)],
    )(*args)


def _fold(lp):
    s = lp['gamma'] / jnp.sqrt(lp['var'] + 1e-5)
    t = (lp['b'] - lp['mean']) * s + lp['beta']
    return lp['W'].T, s[None, :], t[None, :]


def _branch_weights(layers, cin):
    w1t, s1, t1 = _fold(layers[0])
    wx = w1t[cin - 3:, :]
    w2t, s2, t2 = _fold(layers[1])
    w3t, s3, t3 = _fold(layers[2])
    return (wx, s1, t1, w2t, s2, t2, w3t, s3, t3)


def kernel(xyz, params):
    B, _, N = xyz.shape
    S1, S2 = 512, 128

    pts = xyz[:, :3, :]
    xs1, ys1, zs1 = pts[:, 0], pts[:, 1], pts[:, 2]
    feat6 = jnp.concatenate([jnp.transpose(xyz[:, 3:6, :], (0, 2, 1)),
                             jnp.transpose(pts, (0, 2, 1))], axis=-1)
    feat6 = feat6.reshape(B * N, 6)

    sa1 = params['sa1']
    w1ts = [_fold(br[0])[0] for br in sa1]
    A0, A1, A2 = _pre3(feat6, *w1ts)
    As1 = [a.reshape(B, N, a.shape[1]) for a in (A0, A1, A2)]
    nx, ny, nz = _fps(xs1, ys1, zs1, S1)
    cents1 = (nx[:, :, None], ny[:, :, None], nz[:, :, None])
    p3 = lambda a: a.reshape(B, 1, N)
    br1 = [_branch_weights(br, 6) for br in sa1]
    outs1 = _group3(cents1, p3(xs1), p3(ys1), p3(zs1), As1, br1,
                    (16, 32, 128), (0.1, 0.2, 0.4), 32)
    l1_points = jnp.concatenate(outs1, axis=-1)

    sa2 = params['sa2']
    l1_xyz = jnp.stack([nx, ny, nz], axis=-1)
    feat323 = jnp.concatenate([l1_points, l1_xyz], axis=-1).reshape(B * S1, 323)
    w2ts = [_fold(br[0])[0] for br in sa2]
    B0, B1, B2 = _pre3(feat323, *w2ts)
    As2 = [a.reshape(B, S1, a.shape[1]) for a in (B0, B1, B2)]
    mx, my, mz = _fps(nx, ny, nz, S2)
    cents2 = (mx[:, :, None], my[:, :, None], mz[:, :, None])
    q3 = lambda a: a.reshape(B, 1, S1)
    br2 = [_branch_weights(br, 323) for br in sa2]
    outs2 = _group3(cents2, q3(nx), q3(ny), q3(nz), As2, br2,
                    (32, 64, 128), (0.2, 0.4, 0.8), 32)
    l2_points = jnp.concatenate(outs2, axis=-1)

    l2xy = jnp.stack([mx, my], axis=-1)
    gp = jnp.concatenate([l2xy, l2_points], axis=-1).reshape(B * S2, 642)
    sa3 = params['sa3']
    w1, s1, t1 = _fold(sa3[0])
    w2, s2, t2 = _fold(sa3[1])
    w3, s3, t3 = _fold(sa3[2])
    heads = ['g2', 'b1', 'b2', 'b3', 'b4', 'b5']
    cls = params['cls']
    wa = jnp.stack([cls[h][0]['W'].T for h in heads], 0)
    sa_ = jnp.stack([_fold(cls[h][0])[1][0] for h in heads], 0)
    ta_ = jnp.stack([_fold(cls[h][0])[2][0] for h in heads], 0)
    wb = jnp.stack([cls[h][1]['W'].T for h in heads], 0)
    sb_ = jnp.stack([_fold(cls[h][1])[1][0] for h in heads], 0)
    tb_ = jnp.stack([_fold(cls[h][1])[2][0] for h in heads], 0)
    wc = jnp.stack([cls[h][2]['W'].T for h in heads], 0)
    bc = jnp.stack([cls[h][2]['b'] for h in heads], 0)
    x, x_g2, x_b = _sa3_heads(gp, w1, s1, t1, w2, s2, t2, w3, s3, t3,
                              wa, sa_, ta_, wb, sb_, tb_, wc, bc, B, S2)

    l3_points = x[:, :, None]
    g = cls['g2']
    g_w = (g[0]['W'], g[1]['W'], g[2]['W'])
    bkeys = heads[1:]
    b_w = tuple(jnp.transpose(jnp.stack([cls[k][i]['W'] for k in bkeys], 0),
                              (1, 0, 2)) for i in (0, 1, 2))
    return (x_g2, l3_points, x_b, g_w, b_w)


# speedup vs baseline: 27.8120x; 1.0454x over previous
import functools

import jax
import jax.numpy as jnp
import numpy as np
from jax.experimental import pallas as pl

_F32 = jnp.float32


def _dotf(a, b):
    return jax.lax.dot_general(a, b, (((1,), (0,)), ((), ())),
                               preferred_element_type=_F32)


def _fps_body(npoint, xs_ref, ys_ref, zs_ref, ox_ref, oy_ref, oz_ref):
    B, N = xs_ref.shape
    xs = xs_ref[...]
    ys = ys_ref[...]
    zs = zs_ref[...]
    lane = jax.lax.broadcasted_iota(jnp.int32, (B, N), 1)
    col = jax.lax.broadcasted_iota(jnp.int32, (B, npoint), 1)

    def step(i, carry):
        dist, far, ax, ay, az = carry
        oh = lane == far
        cx = jnp.sum(jnp.where(oh, xs, 0.0), axis=1, keepdims=True)
        cy = jnp.sum(jnp.where(oh, ys, 0.0), axis=1, keepdims=True)
        cz = jnp.sum(jnp.where(oh, zs, 0.0), axis=1, keepdims=True)
        dx = xs - cx
        dy = ys - cy
        dz = zs - cz
        d = dx * dx + dy * dy + dz * dz
        dist = jnp.minimum(dist, d)
        mv = jnp.max(dist, axis=1, keepdims=True)
        fnew = jnp.min(jnp.where(dist == mv, lane, N), axis=1, keepdims=True)
        sel = col == i
        ax = jnp.where(sel, cx, ax)
        ay = jnp.where(sel, cy, ay)
        az = jnp.where(sel, cz, az)
        return dist, fnew, ax, ay, az

    init = (jnp.full((B, N), 1e10, _F32),
            jnp.zeros((B, 1), jnp.int32),
            jnp.zeros((B, npoint), _F32),
            jnp.zeros((B, npoint), _F32),
            jnp.zeros((B, npoint), _F32))
    _, _, ax, ay, az = jax.lax.fori_loop(0, npoint, step, init)
    ox_ref[...] = ax
    oy_ref[...] = ay
    oz_ref[...] = az


def _fps(xs, ys, zs, npoint):
    B, N = xs.shape
    full2 = lambda shp: pl.BlockSpec(shp, lambda: tuple(0 for _ in shp))
    return pl.pallas_call(
        functools.partial(_fps_body, npoint),
        in_specs=[full2((B, N))] * 3,
        out_specs=[full2((B, npoint))] * 3,
        out_shape=[jax.ShapeDtypeStruct((B, npoint), _F32)] * 3,
    )(xs, ys, zs)


def _pre3_body(f_ref, w0_ref, w1_ref, w2_ref, o0_ref, o1_ref, o2_ref):
    f = f_ref[...]
    o0_ref[...] = _dotf(f, w0_ref[...])
    o1_ref[...] = _dotf(f, w1_ref[...])
    o2_ref[...] = _dotf(f, w2_ref[...])


def _pre3(feat, w0, w1, w2):
    M = feat.shape[0]
    full = lambda shp: pl.BlockSpec(shp, lambda: tuple(0 for _ in shp))
    return pl.pallas_call(
        _pre3_body,
        in_specs=[full(feat.shape), full(w0.shape), full(w1.shape), full(w2.shape)],
        out_specs=[full((M, w0.shape[1])), full((M, w1.shape[1])), full((M, w2.shape[1]))],
        out_shape=[jax.ShapeDtypeStruct((M, w.shape[1]), _F32) for w in (w0, w1, w2)],
    )(feat, w0, w1, w2)


def _group3_body(Ks, r2s, *refs):
    (cx_ref, cy_ref, cz_ref, xs_ref, ys_ref, zs_ref) = refs[:6]
    a_refs = refs[6:9]
    w_refs = [refs[9 + 9 * i: 9 + 9 * (i + 1)] for i in range(3)]
    o_refs = refs[36:39]
    Ts = cx_ref.shape[1]
    N = xs_ref.shape[2]
    cx = cx_ref[0]
    cy = cy_ref[0]
    cz = cz_ref[0]
    dx = cx - xs_ref[0]
    dy = cy - ys_ref[0]
    dz = cz - zs_ref[0]
    sq = dx * dx + dy * dy + dz * dz
    for br in range(3):
        K = Ks[br]
        r2 = r2s[br]
        a_ref = a_refs[br]
        (wx_ref, s1_ref, t1_ref, w2_ref, s2_ref, t2_ref,
         w3_ref, s3_ref, t3_ref) = w_refs[br]
        C1 = a_ref.shape[2]
        cond = sq <= r2
        condi = jnp.where(cond, 1, 0).astype(jnp.int32)
        c = condi
        d = 1
        while d < N:
            c = c + jnp.concatenate(
                [jnp.zeros((Ts, d), jnp.int32), c[:, :N - d]], axis=1)
            d *= 2
        count = c[:, N - 1:N]
        rankm = jnp.where(cond, c - condi, K)
        kio = jax.lax.broadcasted_iota(jnp.int32, (Ts, K, N), 1)
        P = (rankm[:, None, :] == kio).astype(_F32)
        H = _dotf(P.reshape(Ts * K, N), a_ref[0])
        co = (cx * wx_ref[0:1, :] + cy * wx_ref[1:2, :]
              + cz * wx_ref[2:3, :])
        coK = jnp.broadcast_to(co[:, None, :], (Ts, K, C1)).reshape(Ts * K, C1)
        h = jax.nn.relu((H - coK) * s1_ref[...] + t1_ref[...])
        h = jax.nn.relu(_dotf(h, w2_ref[...]) * s2_ref[...] + t2_ref[...])
        h = jax.nn.relu(_dotf(h, w3_ref[...]) * s3_ref[...] + t3_ref[...])
        C3 = h.shape[1]
        h = h.reshape(Ts, K, C3)
        kio2 = jax.lax.broadcasted_iota(jnp.int32, (Ts, K, 1), 1)
        valid = kio2 < count[:, :, None]
        o_refs[br][0] = jnp.max(jnp.where(valid, h, -jnp.inf), axis=1)


def _group3(cents, xs3, ys3, zs3, As, branches, Ks, radii, Ts):
    cx3, cy3, cz3 = cents
    B, S, _ = cx3.shape
    N = xs3.shape[2]
    r2s = tuple(np.float32(r ** 2) for r in radii)
    cspec = pl.BlockSpec((1, Ts, 1), lambda b, j: (b, j, 0))
    pspec = pl.BlockSpec((1, 1, N), lambda b, j: (b, 0, 0))
    wspec = lambda w: pl.BlockSpec(w.shape, lambda b, j: tuple(0 for _ in w.shape))
    in_specs = [cspec, cspec, cspec, pspec, pspec, pspec]
    args = [cx3, cy3, cz3, xs3, ys3, zs3]
    for A in As:
        in_specs.append(pl.BlockSpec((1, N, A.shape[2]), lambda b, j: (b, 0, 0)))
        args.append(A)
    C3s = []
    for br, lws in enumerate(branches):
        for w in lws:
            in_specs.append(wspec(w))
            args.append(w)
        C3s.append(lws[6].shape[1])
    out_specs = [pl.BlockSpec((1, Ts, C3), lambda b, j: (b, j, 0)) for C3 in C3s]
    out_shape = [jax.ShapeDtypeStruct((B, S, C3), _F32) for C3 in C3s]
    return pl.pallas_call(
        functools.partial(_group3_body, Ks, r2s),
        grid=(B, S // Ts),
        in_specs=in_specs,
        out_specs=out_specs,
        out_shape=out_shape,
    )(*args)


def _sa3_body(B, S, g_ref, w1_ref, s1_ref, t1_ref, w2_ref, s2_ref, t2_ref,
              w3_ref, s3_ref, t3_ref, wa_ref, sa_ref, ta_ref, wb_ref,
              sb_ref, tb_ref, wc_ref, bc_ref, ox_ref, og_ref, ob_ref):
    h = jax.nn.relu(_dotf(g_ref[...], w1_ref[...]) * s1_ref[...] + t1_ref[...])
    h = jax.nn.relu(_dotf(h, w2_ref[...]) * s2_ref[...] + t2_ref[...])
    h = jax.nn.relu(_dotf(h, w3_ref[...]) * s3_ref[...] + t3_ref[...])
    C = h.shape[1]
    x = jnp.max(h.reshape(B, S, C), axis=1)
    ox_ref[...] = x
    outs = []
    for hd in range(6):
        a = jax.nn.relu(_dotf(x, wa_ref[hd]) * sa_ref[hd:hd + 1, :]
                        + ta_ref[hd:hd + 1, :])
        a = jax.nn.relu(_dotf(a, wb_ref[hd]) * sb_ref[hd:hd + 1, :]
                        + tb_ref[hd:hd + 1, :])
        lg = _dotf(a, wc_ref[hd]) + bc_ref[hd:hd + 1, :]
        m = jnp.max(lg, axis=-1, keepdims=True)
        sh = lg - m
        outs.append(sh - jnp.log(jnp.sum(jnp.exp(sh), axis=-1, keepdims=True)))
    og_ref[...] = outs[0]
    M = outs[1]
    for o in outs[2:]:
        M = jnp.maximum(M, o)
    e = jnp.zeros_like(M)
    for o in outs[1:]:
        e = e + jnp.exp(o - M)
    ob_ref[...] = jnp.log(e) + M


def _sa3_heads(gp, w1, s1, t1, w2, s2, t2, w3, s3, t3,
               wa, sa, ta, wb, sb, tb, wc, bc, B, S):
    full = lambda shp: pl.BlockSpec(shp, lambda: tuple(0 for _ in shp))
    args = (gp, w1, s1, t1, w2, s2, t2, w3, s3, t3, wa, sa, ta, wb, sb, tb, wc, bc)
    C = w3.shape[1]
    NC = wc.shape[2]
    return pl.pallas_call(
        functools.partial(_sa3_body, B, S),
        in_specs=[full(a.shape) for a in args],
        out_specs=[full((B, C)), full((B, NC)), full((B, NC))],
        out_shape=[jax.ShapeDtypeStruct((B, C), _F32),
                   jax.ShapeDtypeStruct((B, NC), _F32),
                   jax.ShapeDtypeStruct((B, NC), _F32)],
    )(*args)


def _fold(lp):
    s = lp['gamma'] / jnp.sqrt(lp['var'] + 1e-5)
    t = (lp['b'] - lp['mean']) * s + lp['beta']
    return lp['W'].T, s[None, :], t[None, :]


def _branch_weights(layers, cin):
    w1t, s1, t1 = _fold(layers[0])
    wx = w1t[cin - 3:, :]
    w2t, s2, t2 = _fold(layers[1])
    w3t, s3, t3 = _fold(layers[2])
    return (wx, s1, t1, w2t, s2, t2, w3t, s3, t3)


def kernel(xyz, params):
    B, _, N = xyz.shape
    S1, S2 = 512, 128

    pts = xyz[:, :3, :]
    xs1, ys1, zs1 = pts[:, 0], pts[:, 1], pts[:, 2]
    feat6 = jnp.concatenate([jnp.transpose(xyz[:, 3:6, :], (0, 2, 1)),
                             jnp.transpose(pts, (0, 2, 1))], axis=-1)
    feat6 = feat6.reshape(B * N, 6)

    sa1 = params['sa1']
    w1ts = [_fold(br[0])[0] for br in sa1]
    A0, A1, A2 = _pre3(feat6, *w1ts)
    As1 = [a.reshape(B, N, a.shape[1]) for a in (A0, A1, A2)]
    nx, ny, nz = _fps(xs1, ys1, zs1, S1)
    cents1 = (nx[:, :, None], ny[:, :, None], nz[:, :, None])
    p3 = lambda a: a.reshape(B, 1, N)
    br1 = [_branch_weights(br, 6) for br in sa1]
    outs1 = _group3(cents1, p3(xs1), p3(ys1), p3(zs1), As1, br1,
                    (16, 32, 128), (0.1, 0.2, 0.4), 64)
    l1_points = jnp.concatenate(outs1, axis=-1)

    sa2 = params['sa2']
    l1_xyz = jnp.stack([nx, ny, nz], axis=-1)
    feat323 = jnp.concatenate([l1_points, l1_xyz], axis=-1).reshape(B * S1, 323)
    w2ts = [_fold(br[0])[0] for br in sa2]
    B0, B1, B2 = _pre3(feat323, *w2ts)
    As2 = [a.reshape(B, S1, a.shape[1]) for a in (B0, B1, B2)]
    mx, my, mz = _fps(nx, ny, nz, S2)
    cents2 = (mx[:, :, None], my[:, :, None], mz[:, :, None])
    q3 = lambda a: a.reshape(B, 1, S1)
    br2 = [_branch_weights(br, 323) for br in sa2]
    outs2 = _group3(cents2, q3(nx), q3(ny), q3(nz), As2, br2,
                    (32, 64, 128), (0.2, 0.4, 0.8), 64)
    l2_points = jnp.concatenate(outs2, axis=-1)

    l2xy = jnp.stack([mx, my], axis=-1)
    gp = jnp.concatenate([l2xy, l2_points], axis=-1).reshape(B * S2, 642)
    sa3 = params['sa3']
    w1, s1, t1 = _fold(sa3[0])
    w2, s2, t2 = _fold(sa3[1])
    w3, s3, t3 = _fold(sa3[2])
    heads = ['g2', 'b1', 'b2', 'b3', 'b4', 'b5']
    cls = params['cls']
    wa = jnp.stack([cls[h][0]['W'].T for h in heads], 0)
    sa_ = jnp.stack([_fold(cls[h][0])[1][0] for h in heads], 0)
    ta_ = jnp.stack([_fold(cls[h][0])[2][0] for h in heads], 0)
    wb = jnp.stack([cls[h][1]['W'].T for h in heads], 0)
    sb_ = jnp.stack([_fold(cls[h][1])[1][0] for h in heads], 0)
    tb_ = jnp.stack([_fold(cls[h][1])[2][0] for h in heads], 0)
    wc = jnp.stack([cls[h][2]['W'].T for h in heads], 0)
    bc = jnp.stack([cls[h][2]['b'] for h in heads], 0)
    x, x_g2, x_b = _sa3_heads(gp, w1, s1, t1, w2, s2, t2, w3, s3, t3,
                              wa, sa_, ta_, wb, sb_, tb_, wc, bc, B, S2)

    l3_points = x[:, :, None]
    g = cls['g2']
    g_w = (g[0]['W'], g[1]['W'], g[2]['W'])
    bkeys = heads[1:]
    b_w = tuple(jnp.transpose(jnp.stack([cls[k][i]['W'] for k in bkeys], 0),
                              (1, 0, 2)) for i in (0, 1, 2))
    return (x_g2, l3_points, x_b, g_w, b_w)
